# baseline (speedup 1.0000x reference)
import functools

import jax
import jax.numpy as jnp
from jax import lax
from jax.experimental import pallas as pl
from jax.experimental.pallas import tpu as pltpu
from jax.experimental.pallas import tpu_sc as plsc

N = 10000
E = 320000
D = 128
PC = 8
L = 16

NC = 2
NS = 16
NW = NC * NS

CB = 128
NCH = E // CB
BASE_CH = NCH // NW
EXTRA = NCH - BASE_CH * NW

ROWS_TC = 1000


def _tables_body(x_ref, posp_ref, wxi_ref, wx_ref, wp_ref, bias_ref,
                 a_ref, b_ref):
    pw = jnp.dot(posp_ref[...], wp_ref[...],
                 preferred_element_type=jnp.float32)
    xw = jnp.dot(x_ref[...], wx_ref[...],
                 preferred_element_type=jnp.float32)
    xi = jnp.dot(x_ref[...], wxi_ref[...],
                 preferred_element_type=jnp.float32)
    a_ref[...] = xw + pw
    b_ref[...] = xi - pw + bias_ref[...]


def _compute_tables(x, posp, w_xi, w_x, w_p, bias):
    return pl.pallas_call(
        _tables_body,
        grid=(N // ROWS_TC,),
        in_specs=[
            pl.BlockSpec((ROWS_TC, D), lambda i: (i, 0)),
            pl.BlockSpec((ROWS_TC, PC), lambda i: (i, 0)),
            pl.BlockSpec((D, D), lambda i: (0, 0)),
            pl.BlockSpec((D, D), lambda i: (0, 0)),
            pl.BlockSpec((PC, D), lambda i: (0, 0)),
            pl.BlockSpec((1, D), lambda i: (0, 0)),
        ],
        out_specs=[
            pl.BlockSpec((ROWS_TC, D), lambda i: (i, 0)),
            pl.BlockSpec((ROWS_TC, D), lambda i: (i, 0)),
        ],
        out_shape=[
            jax.ShapeDtypeStruct((N, D), jnp.float32),
            jax.ShapeDtypeStruct((N, D), jnp.float32),
        ],
    )(x, posp, w_xi, w_x, w_p, bias)


def _edge_body(a_hbm, b_hbm, nbr_hbm, nbb_hbm, out_hbm,
               idx_a, idx_b, rows_a, rows_b, sem):
    wid = lax.axis_index("s") * NC + lax.axis_index("c")
    nch = BASE_CH + jnp.where(wid < EXTRA, 1, 0)

    def chunk(g, carry):
        cid = wid + g * NW
        pltpu.sync_copy(nbr_hbm.at[cid], idx_a)
        pltpu.sync_copy(nbb_hbm.at[cid], idx_b)
        ca = pltpu.async_copy(a_hbm.at[idx_a], rows_a, sem)
        ca.wait()
        cb = pltpu.async_copy(b_hbm.at[idx_b], rows_a, sem, add=True)
        cb.wait()
        pltpu.sync_copy(rows_a, out_hbm.at[pl.ds(cid * CB, CB)])
        return carry

    lax.fori_loop(0, nch, chunk, 0)


@functools.lru_cache(maxsize=1)
def _edge_kernel():
    return functools.partial(
        pl.kernel,
        mesh=plsc.VectorSubcoreMesh(core_axis_name="c", subcore_axis_name="s",
                                    num_cores=NC, num_subcores=NS),
        out_type=jax.ShapeDtypeStruct((E, D), jnp.float32),
        scratch_types=[
            pltpu.VMEM((CB,), jnp.int32),
            pltpu.VMEM((CB,), jnp.int32),
            pltpu.VMEM((CB, D), jnp.float32),
            pltpu.VMEM((CB, D), jnp.float32),
            pltpu.SemaphoreType.DMA,
        ],
    )(_edge_body)


def kernel(pos, x, neighbors, neighbor_batch, W_xi, b_xi, W_xn, b_xn):
    w_p = jnp.zeros((PC, D), jnp.float32).at[:3].set(W_xn[:3])
    w_x = W_xn[3:]
    posp = jnp.pad(pos, ((0, 0), (0, PC - 3)))
    bias = (b_xi + b_xn).reshape(1, D)
    a_tab, b_tab = _compute_tables(x, posp, W_xi, w_x, w_p, bias)
    nbr2d = neighbors.reshape(NCH, CB)
    nbb2d = neighbor_batch.reshape(NCH, CB)
    return _edge_kernel()(a_tab, b_tab, nbr2d, nbb2d)


# speedup vs baseline: 1.5267x; 1.5267x over previous
import functools

import jax
import jax.numpy as jnp
from jax import lax
from jax.experimental import pallas as pl
from jax.experimental.pallas import tpu as pltpu
from jax.experimental.pallas import tpu_sc as plsc

N = 10000
E = 320000
D = 128
PC = 8

NC = 2
NS = 16
NW = NC * NS

CB = 80
NCH = E // CB
CPW = NCH // NW
NSLOT = 5

ROWS_TC = 1000


def _tables_body(x_ref, posp_ref, wxi_ref, wx_ref, wp_ref, bias_ref,
                 a_ref, b_ref):
    pw = jnp.dot(posp_ref[...], wp_ref[...],
                 preferred_element_type=jnp.float32)
    xw = jnp.dot(x_ref[...], wx_ref[...],
                 preferred_element_type=jnp.float32)
    xi = jnp.dot(x_ref[...], wxi_ref[...],
                 preferred_element_type=jnp.float32)
    a_ref[...] = xw + pw
    b_ref[...] = xi - pw + bias_ref[...]


def _compute_tables(x, posp, w_xi, w_x, w_p, bias):
    return pl.pallas_call(
        _tables_body,
        grid=(N // ROWS_TC,),
        in_specs=[
            pl.BlockSpec((ROWS_TC, D), lambda i: (i, 0)),
            pl.BlockSpec((ROWS_TC, PC), lambda i: (i, 0)),
            pl.BlockSpec((D, D), lambda i: (0, 0)),
            pl.BlockSpec((D, D), lambda i: (0, 0)),
            pl.BlockSpec((PC, D), lambda i: (0, 0)),
            pl.BlockSpec((1, D), lambda i: (0, 0)),
        ],
        out_specs=[
            pl.BlockSpec((ROWS_TC, D), lambda i: (i, 0)),
            pl.BlockSpec((ROWS_TC, D), lambda i: (i, 0)),
        ],
        out_shape=[
            jax.ShapeDtypeStruct((N, D), jnp.float32),
            jax.ShapeDtypeStruct((N, D), jnp.float32),
        ],
    )(x, posp, w_xi, w_x, w_p, bias)


def _edge_body(a_hbm, b_hbm, nbr_hbm, nbb_hbm, out_hbm,
               idx_a, idx_b, rows0, rows1, rows2, rows3, rows4,
               sem0, sem1, sem2, sem3, sem4):
    rows = (rows0, rows1, rows2, rows3, rows4)
    sems = (sem0, sem1, sem2, sem3, sem4)
    wid = lax.axis_index("s") * NC + lax.axis_index("c")
    ch0 = wid * CPW

    pltpu.sync_copy(nbr_hbm.at[wid], idx_a)
    pltpu.sync_copy(nbb_hbm.at[wid], idx_b)

    def start_ga(g, s):
        pltpu.async_copy(a_hbm.at[idx_a.at[g]], rows[s], sems[s])

    def wait_slot(g, s):
        pltpu.make_async_copy(a_hbm.at[idx_a.at[g]], rows[s], sems[s]).wait()

    def start_gb(g, s):
        pltpu.async_copy(b_hbm.at[idx_b.at[g]], rows[s], sems[s], add=True)

    def _out_slice(g):
        off = pl.multiple_of((ch0 + g) * CB, 8)
        return out_hbm.at[pl.ds(off, CB)]

    def start_w(g, s):
        pltpu.async_copy(rows[s], _out_slice(g), sems[s])

    def wait_w(g, s):
        pltpu.make_async_copy(rows[s], _out_slice(g), sems[s]).wait()

    def stages(g, steady):
        b = g % NSLOT if isinstance(g, int) else None
        if steady:
            wait_w(g - NSLOT, b)
        start_ga(g, b)
        if isinstance(g, int) and g < 1:
            return
        wait_slot(g - 1, (g - 1) % NSLOT)
        start_gb(g - 1, (g - 1) % NSLOT)
        if isinstance(g, int) and g < 2:
            return
        wait_slot(g - 2, (g - 2) % NSLOT)
        start_w(g - 2, (g - 2) % NSLOT)

    for g in range(NSLOT):
        stages(g, steady=False)

    def group(g0, carry):
        for b in range(NSLOT):
            g = g0 * NSLOT + b
            wait_w(g - NSLOT, b)
            start_ga(g, b)
            wait_slot(g - 1, (b - 1) % NSLOT)
            start_gb(g - 1, (b - 1) % NSLOT)
            wait_slot(g - 2, (b - 2) % NSLOT)
            start_w(g - 2, (b - 2) % NSLOT)
        return carry

    lax.fori_loop(1, CPW // NSLOT, group, 0)

    g = CPW
    wait_slot(g - 1, (g - 1) % NSLOT)
    start_gb(g - 1, (g - 1) % NSLOT)
    wait_slot(g - 2, (g - 2) % NSLOT)
    start_w(g - 2, (g - 2) % NSLOT)
    wait_slot(g - 1, (g - 1) % NSLOT)
    start_w(g - 1, (g - 1) % NSLOT)
    for t in range(NSLOT):
        wait_w(CPW - NSLOT + t, (CPW - NSLOT + t) % NSLOT)


@functools.lru_cache(maxsize=1)
def _edge_kernel():
    return functools.partial(
        pl.kernel,
        mesh=plsc.VectorSubcoreMesh(core_axis_name="c", subcore_axis_name="s",
                                    num_cores=NC, num_subcores=NS),
        out_type=jax.ShapeDtypeStruct((E, D), jnp.float32),
        scratch_types=(
            [pltpu.VMEM((CPW, CB), jnp.int32)] * 2
            + [pltpu.VMEM((CB, D), jnp.float32)] * NSLOT
            + [pltpu.SemaphoreType.DMA] * NSLOT
        ),
    )(_edge_body)


def kernel(pos, x, neighbors, neighbor_batch, W_xi, b_xi, W_xn, b_xn):
    w_p = jnp.zeros((PC, D), jnp.float32).at[:3].set(W_xn[:3])
    w_x = W_xn[3:]
    posp = jnp.pad(pos, ((0, 0), (0, PC - 3)))
    bias = (b_xi + b_xn).reshape(1, D)
    a_tab, b_tab = _compute_tables(x, posp, W_xi, w_x, w_p, bias)
    nbr3d = neighbors.reshape(NW, CPW, CB)
    nbb3d = neighbor_batch.reshape(NW, CPW, CB)
    return _edge_kernel()(a_tab, b_tab, nbr3d, nbb3d)
